# baseline (speedup 1.0000x reference)
import functools
import math

import jax
import jax.numpy as jnp
from jax import lax
from jax.experimental import pallas as pl
from jax.experimental.pallas import tpu as pltpu
from jax.experimental.pallas import tpu_sc as plsc

N = 10000
E = 320000
D = 128
H = 128

C = 128
NW_STATIC = 32
NCHUNK = E // C
CH_T = NCHUNK // NW_STATIC
EXTRA = NCHUNK - NW_STATIC * CH_T


def _sc_aggregate(x, dst3d, src3d, val3d):
    info = plsc.get_sparse_core_info()
    NC, NS = info.num_cores, info.num_subcores
    rows_per_tile = (N // NS) // 8 * 8
    rem_rows = N - NS * rows_per_tile
    rem_base = NS * rows_per_tile

    mesh = plsc.VectorSubcoreMesh(core_axis_name="c", subcore_axis_name="s")

    @functools.partial(
        pl.kernel,
        mesh=mesh,
        out_type=(
            jax.ShapeDtypeStruct((N, D), jnp.float32),
            jax.ShapeDtypeStruct((N, D), jnp.float32),
        ),
        scratch_types=[
            pltpu.VMEM((1, 1, C), jnp.int32),
            pltpu.VMEM((1, 1, C), jnp.int32),
            pltpu.VMEM((1, 1, C), jnp.int32),
            pltpu.VMEM((1, 1, C), jnp.int32),
            pltpu.VMEM((1, 1, C), jnp.int32),
            pltpu.VMEM((1, 1, C), jnp.int32),
            pltpu.VMEM((1, 1, C), jnp.float32),
            pltpu.VMEM((1, 1, C), jnp.float32),
            pltpu.VMEM((1, 1, C), jnp.float32),
            pltpu.VMEM((C, D), jnp.float32),
            pltpu.VMEM((C, D), jnp.float32),
            pltpu.VMEM((C, D), jnp.float32),
            pltpu.VMEM_SHARED((N, D), jnp.float32),
            pltpu.SemaphoreType.DMA,
            pltpu.SemaphoreType.DMA,
            pltpu.SemaphoreType.DMA,
            pltpu.SemaphoreType.DMA,
            pltpu.SemaphoreType.DMA,
            pltpu.SemaphoreType.DMA,
            pltpu.SemaphoreType.DMA,
            pltpu.SemaphoreType.DMA,
            pltpu.SemaphoreType.DMA,
        ],
    )
    def agg_kernel(x_hbm, dst_hbm, src_hbm, val_hbm, out0, out1,
                   db0, db1, db2, sb0, sb1, sb2, vb0, vb1, vb2,
                   rg0, rg1, rg2, acc_sh,
                   gsem0, gsem1, gsem2, ssem0, ssem1, ssem2,
                   isem0, isem1, isem2):
        cid = lax.axis_index("c")
        sid = lax.axis_index("s")
        wid = sid * NC + cid
        base_ch = wid * CH_T
        db = (db0, db1, db2)
        sb = (sb0, sb1, sb2)
        vb = (vb0, vb1, vb2)
        rg = (rg0, rg1, rg2)
        gsem = (gsem0, gsem1, gsem2)
        ssem = (ssem0, ssem1, ssem2)
        isem = (isem0, isem1, isem2)

        def idx_start(i, p):
            sl = pl.ds(base_ch + i, 1)
            pltpu.async_copy(dst_hbm.at[sl], db[p], isem[p])
            pltpu.async_copy(src_hbm.at[sl], sb[p], isem[p])
            pltpu.async_copy(val_hbm.at[sl], vb[p], isem[p])

        def idx_wait(i, p):
            sl = pl.ds(base_ch + i, 1)
            pltpu.make_async_copy(dst_hbm.at[sl], db[p], isem[p]).wait()
            pltpu.make_async_copy(src_hbm.at[sl], sb[p], isem[p]).wait()
            pltpu.make_async_copy(val_hbm.at[sl], vb[p], isem[p]).wait()

        def scale_rows(rg_ref, vb_ref):
            def scale(g, _):
                val16 = vb_ref[0, 0, pl.ds(g * 16, 16)]
                for l in range(16):
                    v = val16[l]
                    j = g * 16 + l
                    for k in range(D // 16):
                        sl2 = pl.ds(16 * k, 16)
                        rg_ref[j, sl2] = rg_ref[j, sl2] * v
                return 0
            lax.fori_loop(0, C // 16, scale, 0)

        idx_start(0, 0)

        def zrow(r, _):
            for k in range(D // 16):
                rg0[r, pl.ds(16 * k, 16)] = jnp.zeros((16,), jnp.float32)
            return 0
        lax.fori_loop(0, C, zrow, 0)
        base = sid * rows_per_tile
        nfull = rows_per_tile // C
        ztail = rows_per_tile - nfull * C
        for i in range(nfull):
            pltpu.async_copy(rg0, acc_sh.at[pl.ds(base + i * C, C)], ssem0)
        pltpu.async_copy(rg0.at[pl.ds(0, ztail)],
                         acc_sh.at[pl.ds(base + nfull * C, ztail)], ssem0)

        @pl.when(sid == 0)
        def _():
            pltpu.async_copy(rg0.at[pl.ds(0, rem_rows)],
                             acc_sh.at[pl.ds(rem_base, rem_rows)], ssem0)

        for i in range(nfull):
            pltpu.make_async_copy(
                rg0, acc_sh.at[pl.ds(base + i * C, C)], ssem0).wait()
        pltpu.make_async_copy(
            rg0.at[pl.ds(0, ztail)],
            acc_sh.at[pl.ds(base + nfull * C, ztail)], ssem0).wait()

        @pl.when(sid == 0)
        def _():
            pltpu.make_async_copy(
                rg0.at[pl.ds(0, rem_rows)],
                acc_sh.at[pl.ds(rem_base, rem_rows)], ssem0).wait()

        idx_wait(0, 0)
        pltpu.async_copy(x_hbm.at[sb0.at[0, 0]], rg0, gsem0)
        idx_start(1, 1)

        plsc.subcore_barrier()

        def triple(t, _):
            for m in range(3):
                i = 3 * t + m
                m1 = (m + 1) % 3
                m2 = (m + 2) % 3

                @pl.when(i + 1 < CH_T)
                def _():
                    idx_wait(i + 1, m1)
                    pltpu.async_copy(x_hbm.at[sb[m1].at[0, 0]],
                                     rg[m1], gsem[m1])

                pltpu.make_async_copy(x_hbm.at[sb[m].at[0, 0]],
                                      rg[m], gsem[m]).wait()
                scale_rows(rg[m], vb[m])
                pltpu.async_copy(rg[m], acc_sh.at[db[m].at[0, 0]],
                                 ssem[m], add=True)

                @pl.when(i >= 1)
                def _():
                    pltpu.make_async_copy(
                        rg[m2], acc_sh.at[db[m2].at[0, 0]], ssem[m2]).wait()

                @pl.when(i + 2 < CH_T)
                def _():
                    idx_start(i + 2, m2)
            return 0
        lax.fori_loop(0, CH_T // 3, triple, 0)

        _mlast = (CH_T - 1) % 3
        pltpu.make_async_copy(rg[_mlast],
                              acc_sh.at[db[_mlast].at[0, 0]],
                              ssem[_mlast]).wait()

        @pl.when(wid < EXTRA)
        def _():
            sl = pl.ds(NW_STATIC * CH_T + wid, 1)
            pltpu.sync_copy(dst_hbm.at[sl], db0)
            pltpu.sync_copy(src_hbm.at[sl], sb0)
            pltpu.sync_copy(val_hbm.at[sl], vb0)
            pltpu.async_copy(x_hbm.at[sb0.at[0, 0]], rg0, gsem0).wait()
            scale_rows(rg0, vb0)
            pltpu.sync_copy(rg0, acc_sh.at[db0.at[0, 0]], add=True)

        plsc.subcore_barrier()

        def copy_out(out_ref):
            for i in range(nfull):
                pltpu.async_copy(acc_sh.at[pl.ds(base + i * C, C)],
                                 out_ref.at[pl.ds(base + i * C, C)], ssem0)
            pltpu.async_copy(acc_sh.at[pl.ds(base + nfull * C, ztail)],
                             out_ref.at[pl.ds(base + nfull * C, ztail)],
                             ssem0)

            @pl.when(sid == 0)
            def _():
                pltpu.async_copy(acc_sh.at[pl.ds(rem_base, rem_rows)],
                                 out_ref.at[pl.ds(rem_base, rem_rows)],
                                 ssem0)

            for i in range(nfull):
                pltpu.make_async_copy(
                    acc_sh.at[pl.ds(base + i * C, C)],
                    out_ref.at[pl.ds(base + i * C, C)], ssem0).wait()
            pltpu.make_async_copy(
                acc_sh.at[pl.ds(base + nfull * C, ztail)],
                out_ref.at[pl.ds(base + nfull * C, ztail)], ssem0).wait()

            @pl.when(sid == 0)
            def _():
                pltpu.make_async_copy(
                    acc_sh.at[pl.ds(rem_base, rem_rows)],
                    out_ref.at[pl.ds(rem_base, rem_rows)], ssem0).wait()

        @pl.when(cid == 0)
        def _():
            copy_out(out0)

        @pl.when(cid == 1)
        def _():
            copy_out(out1)

    return agg_kernel(x, dst3d, src3d, val3d)


_BR = 1000
_INV_SQRT2 = 1.0 / math.sqrt(2.0)


def _ln_gelu(h, g, be):
    mu = jnp.mean(h, axis=-1, keepdims=True)
    var = jnp.mean((h - mu) ** 2, axis=-1, keepdims=True)
    h = (h - mu) / jnp.sqrt(var + 1e-5) * g + be
    return 0.5 * h * (1.0 + lax.erf(h * _INV_SQRT2))


def _dense1_body(x_ref, a0_ref, a1_ref, W_ref, b_ref, g_ref, be_ref, o_ref):
    h = x_ref[...] + a0_ref[...] + a1_ref[...]
    h = jnp.dot(h, W_ref[...], preferred_element_type=jnp.float32) + b_ref[...]
    o_ref[...] = _ln_gelu(h, g_ref[...], be_ref[...])


def _dense2_body(x_ref, a0_ref, a1_ref, W2_ref, b2_ref, g2_ref, be2_ref,
                 Wf_ref, bf_ref, o_ref):
    h = x_ref[...] + a0_ref[...] + a1_ref[...]
    h = jnp.dot(h, W2_ref[...], preferred_element_type=jnp.float32) + b2_ref[...]
    h = _ln_gelu(h, g2_ref[...], be2_ref[...])
    o_ref[...] = jnp.dot(h, Wf_ref[...], preferred_element_type=jnp.float32) + bf_ref[...]


def _row_spec():
    return pl.BlockSpec((_BR, D), lambda i: (i, 0))


def _rep_spec(shape):
    return pl.BlockSpec(shape, lambda i: (0,) * len(shape))


def _dense1(x, a0, a1, W, b, g, be):
    return pl.pallas_call(
        _dense1_body,
        grid=(N // _BR,),
        in_specs=[_row_spec(), _row_spec(), _row_spec(),
                  _rep_spec((D, H)), _rep_spec((1, H)), _rep_spec((1, H)),
                  _rep_spec((1, H))],
        out_specs=_row_spec(),
        out_shape=jax.ShapeDtypeStruct((N, H), jnp.float32),
    )(x, a0, a1, W, b.reshape(1, H), g.reshape(1, H), be.reshape(1, H))


def _dense2(x, a0, a1, W2, b2, g2, be2, Wf, bf):
    return pl.pallas_call(
        _dense2_body,
        grid=(N // _BR,),
        in_specs=[_row_spec(), _row_spec(), _row_spec(),
                  _rep_spec((H, H)), _rep_spec((1, H)), _rep_spec((1, H)),
                  _rep_spec((1, H)),
                  _rep_spec((H, D)), _rep_spec((1, D))],
        out_specs=_row_spec(),
        out_shape=jax.ShapeDtypeStruct((N, D), jnp.float32),
    )(x, a0, a1, W2, b2.reshape(1, H), g2.reshape(1, H), be2.reshape(1, H),
      Wf, bf.reshape(1, D))


def kernel(node_features, adj_indices, adj_values, W1, b1, g1, be1,
           W2, b2, g2, be2, Wf, bf):
    dst3d = adj_indices[0].astype(jnp.int32).reshape(NCHUNK, 1, C)
    src3d = adj_indices[1].astype(jnp.int32).reshape(NCHUNK, 1, C)
    val3d = adj_values.reshape(NCHUNK, 1, C)

    a0, a1 = _sc_aggregate(node_features, dst3d, src3d, val3d)
    x1 = _dense1(node_features, a0, a1, W1, b1, g1, be1)
    c0, c1 = _sc_aggregate(x1, dst3d, src3d, val3d)
    return _dense2(x1, c0, c1, W2, b2, g2, be2, Wf, bf)


# speedup vs baseline: 1.0144x; 1.0144x over previous
import functools
import math

import jax
import jax.numpy as jnp
from jax import lax
from jax.experimental import pallas as pl
from jax.experimental.pallas import tpu as pltpu
from jax.experimental.pallas import tpu_sc as plsc

N = 10000
E = 320000
D = 128
H = 128

C = 128
NW_STATIC = 32
NCHUNK = E // C
CH_T = NCHUNK // NW_STATIC
EXTRA = NCHUNK - NW_STATIC * CH_T


def _sc_aggregate(x, dst3d, src3d, val3d):
    info = plsc.get_sparse_core_info()
    NC, NS = info.num_cores, info.num_subcores
    rows_per_tile = (N // NS) // 8 * 8
    rem_rows = N - NS * rows_per_tile
    rem_base = NS * rows_per_tile

    mesh = plsc.VectorSubcoreMesh(core_axis_name="c", subcore_axis_name="s")

    @functools.partial(
        pl.kernel,
        mesh=mesh,
        out_type=(
            jax.ShapeDtypeStruct((N, D), jnp.float32),
            jax.ShapeDtypeStruct((N, D), jnp.float32),
        ),
        scratch_types=[
            pltpu.VMEM((1, 1, C), jnp.int32),
            pltpu.VMEM((1, 1, C), jnp.int32),
            pltpu.VMEM((1, 1, C), jnp.int32),
            pltpu.VMEM((1, 1, C), jnp.int32),
            pltpu.VMEM((1, 1, C), jnp.int32),
            pltpu.VMEM((1, 1, C), jnp.int32),
            pltpu.VMEM((1, 1, C), jnp.float32),
            pltpu.VMEM((1, 1, C), jnp.float32),
            pltpu.VMEM((1, 1, C), jnp.float32),
            pltpu.VMEM((C, D), jnp.float32),
            pltpu.VMEM((C, D), jnp.float32),
            pltpu.VMEM((C, D), jnp.float32),
            pltpu.VMEM_SHARED((N, D), jnp.float32),
            pltpu.SemaphoreType.DMA,
            pltpu.SemaphoreType.DMA,
            pltpu.SemaphoreType.DMA,
            pltpu.SemaphoreType.DMA,
            pltpu.SemaphoreType.DMA,
            pltpu.SemaphoreType.DMA,
            pltpu.SemaphoreType.DMA,
            pltpu.SemaphoreType.DMA,
            pltpu.SemaphoreType.DMA,
        ],
    )
    def agg_kernel(x_hbm, dst_hbm, src_hbm, val_hbm, out0, out1,
                   db0, db1, db2, sb0, sb1, sb2, vb0, vb1, vb2,
                   rg0, rg1, rg2, acc_sh,
                   gsem0, gsem1, gsem2, ssem0, ssem1, ssem2,
                   isem0, isem1, isem2):
        cid = lax.axis_index("c")
        sid = lax.axis_index("s")
        wid = sid * NC + cid
        base_ch = wid * CH_T
        db = (db0, db1, db2)
        sb = (sb0, sb1, sb2)
        vb = (vb0, vb1, vb2)
        rg = (rg0, rg1, rg2)
        gsem = (gsem0, gsem1, gsem2)
        ssem = (ssem0, ssem1, ssem2)
        isem = (isem0, isem1, isem2)

        def idx_start(i, p):
            sl = pl.ds(base_ch + i, 1)
            pltpu.async_copy(dst_hbm.at[sl], db[p], isem[p])
            pltpu.async_copy(src_hbm.at[sl], sb[p], isem[p])
            pltpu.async_copy(val_hbm.at[sl], vb[p], isem[p])

        def idx_wait(i, p):
            sl = pl.ds(base_ch + i, 1)
            pltpu.make_async_copy(dst_hbm.at[sl], db[p], isem[p]).wait()
            pltpu.make_async_copy(src_hbm.at[sl], sb[p], isem[p]).wait()
            pltpu.make_async_copy(val_hbm.at[sl], vb[p], isem[p]).wait()

        def scale_rows(rg_ref, vb_ref):
            def scale(g, _):
                val16 = vb_ref[0, 0, pl.ds(g * 16, 16)]
                for l in range(16):
                    v = val16[l]
                    j = g * 16 + l
                    for k in range(D // 16):
                        sl2 = pl.ds(16 * k, 16)
                        rg_ref[j, sl2] = rg_ref[j, sl2] * v
                return 0
            lax.fori_loop(0, C // 16, scale, 0)

        idx_start(0, 0)

        def zrow(r, _):
            for k in range(D // 16):
                rg0[r, pl.ds(16 * k, 16)] = jnp.zeros((16,), jnp.float32)
            return 0
        lax.fori_loop(0, C, zrow, 0)
        base = sid * rows_per_tile
        nfull = rows_per_tile // C
        ztail = rows_per_tile - nfull * C
        for i in range(nfull):
            pltpu.async_copy(rg0, acc_sh.at[pl.ds(base + i * C, C)], ssem0)
        pltpu.async_copy(rg0.at[pl.ds(0, ztail)],
                         acc_sh.at[pl.ds(base + nfull * C, ztail)], ssem0)

        @pl.when(sid == 0)
        def _():
            pltpu.async_copy(rg0.at[pl.ds(0, rem_rows)],
                             acc_sh.at[pl.ds(rem_base, rem_rows)], ssem0)

        for i in range(nfull):
            pltpu.make_async_copy(
                rg0, acc_sh.at[pl.ds(base + i * C, C)], ssem0).wait()
        pltpu.make_async_copy(
            rg0.at[pl.ds(0, ztail)],
            acc_sh.at[pl.ds(base + nfull * C, ztail)], ssem0).wait()

        @pl.when(sid == 0)
        def _():
            pltpu.make_async_copy(
                rg0.at[pl.ds(0, rem_rows)],
                acc_sh.at[pl.ds(rem_base, rem_rows)], ssem0).wait()

        idx_wait(0, 0)
        pltpu.async_copy(x_hbm.at[sb0.at[0, 0]], rg0, gsem0)
        idx_start(1, 1)

        plsc.subcore_barrier()

        def triple(t, _):
            for m in range(3):
                i = 3 * t + m
                m1 = (m + 1) % 3
                m2 = (m + 2) % 3

                @pl.when(i + 1 < CH_T)
                def _():
                    idx_wait(i + 1, m1)
                    pltpu.async_copy(x_hbm.at[sb[m1].at[0, 0]],
                                     rg[m1], gsem[m1])

                pltpu.make_async_copy(x_hbm.at[sb[m].at[0, 0]],
                                      rg[m], gsem[m]).wait()
                scale_rows(rg[m], vb[m])
                pltpu.async_copy(rg[m], acc_sh.at[db[m].at[0, 0]],
                                 ssem[m], add=True)

                @pl.when(i >= 1)
                def _():
                    pltpu.make_async_copy(
                        rg[m2], acc_sh.at[db[m2].at[0, 0]], ssem[m2]).wait()

                @pl.when(i + 2 < CH_T)
                def _():
                    idx_start(i + 2, m2)
            return 0
        lax.fori_loop(0, CH_T // 3, triple, 0)

        _mlast = (CH_T - 1) % 3
        pltpu.make_async_copy(rg[_mlast],
                              acc_sh.at[db[_mlast].at[0, 0]],
                              ssem[_mlast]).wait()

        @pl.when(wid < EXTRA)
        def _():
            sl = pl.ds(NW_STATIC * CH_T + wid, 1)
            pltpu.sync_copy(dst_hbm.at[sl], db0)
            pltpu.sync_copy(src_hbm.at[sl], sb0)
            pltpu.sync_copy(val_hbm.at[sl], vb0)
            pltpu.async_copy(x_hbm.at[sb0.at[0, 0]], rg0, gsem0).wait()
            scale_rows(rg0, vb0)
            pltpu.sync_copy(rg0, acc_sh.at[db0.at[0, 0]], add=True)

        plsc.subcore_barrier()

        def copy_out(out_ref):
            for i in range(nfull):
                pltpu.async_copy(acc_sh.at[pl.ds(base + i * C, C)],
                                 out_ref.at[pl.ds(base + i * C, C)], ssem0)
            pltpu.async_copy(acc_sh.at[pl.ds(base + nfull * C, ztail)],
                             out_ref.at[pl.ds(base + nfull * C, ztail)],
                             ssem0)

            @pl.when(sid == 0)
            def _():
                pltpu.async_copy(acc_sh.at[pl.ds(rem_base, rem_rows)],
                                 out_ref.at[pl.ds(rem_base, rem_rows)],
                                 ssem0)

            for i in range(nfull):
                pltpu.make_async_copy(
                    acc_sh.at[pl.ds(base + i * C, C)],
                    out_ref.at[pl.ds(base + i * C, C)], ssem0).wait()
            pltpu.make_async_copy(
                acc_sh.at[pl.ds(base + nfull * C, ztail)],
                out_ref.at[pl.ds(base + nfull * C, ztail)], ssem0).wait()

            @pl.when(sid == 0)
            def _():
                pltpu.make_async_copy(
                    acc_sh.at[pl.ds(rem_base, rem_rows)],
                    out_ref.at[pl.ds(rem_base, rem_rows)], ssem0).wait()

        @pl.when(cid == 0)
        def _():
            copy_out(out0)

        @pl.when(cid == 1)
        def _():
            copy_out(out1)

    return agg_kernel(x, dst3d, src3d, val3d)


_BR = 2000
_INV_SQRT2 = 1.0 / math.sqrt(2.0)


def _ln_gelu(h, g, be):
    mu = jnp.mean(h, axis=-1, keepdims=True)
    var = jnp.mean((h - mu) ** 2, axis=-1, keepdims=True)
    h = (h - mu) / jnp.sqrt(var + 1e-5) * g + be
    return 0.5 * h * (1.0 + lax.erf(h * _INV_SQRT2))


def _dense1_body(x_ref, a0_ref, a1_ref, W_ref, b_ref, g_ref, be_ref, o_ref):
    h = x_ref[...] + a0_ref[...] + a1_ref[...]
    h = jnp.dot(h, W_ref[...], preferred_element_type=jnp.float32) + b_ref[...]
    o_ref[...] = _ln_gelu(h, g_ref[...], be_ref[...])


def _dense2_body(x_ref, a0_ref, a1_ref, W2_ref, b2_ref, g2_ref, be2_ref,
                 Wf_ref, bf_ref, o_ref):
    h = x_ref[...] + a0_ref[...] + a1_ref[...]
    h = jnp.dot(h, W2_ref[...], preferred_element_type=jnp.float32) + b2_ref[...]
    h = _ln_gelu(h, g2_ref[...], be2_ref[...])
    o_ref[...] = jnp.dot(h, Wf_ref[...], preferred_element_type=jnp.float32) + bf_ref[...]


def _row_spec():
    return pl.BlockSpec((_BR, D), lambda i: (i, 0))


def _rep_spec(shape):
    return pl.BlockSpec(shape, lambda i: (0,) * len(shape))


def _dense1(x, a0, a1, W, b, g, be):
    return pl.pallas_call(
        _dense1_body,
        grid=(N // _BR,),
        in_specs=[_row_spec(), _row_spec(), _row_spec(),
                  _rep_spec((D, H)), _rep_spec((1, H)), _rep_spec((1, H)),
                  _rep_spec((1, H))],
        out_specs=_row_spec(),
        out_shape=jax.ShapeDtypeStruct((N, H), jnp.float32),
    )(x, a0, a1, W, b.reshape(1, H), g.reshape(1, H), be.reshape(1, H))


def _dense2(x, a0, a1, W2, b2, g2, be2, Wf, bf):
    return pl.pallas_call(
        _dense2_body,
        grid=(N // _BR,),
        in_specs=[_row_spec(), _row_spec(), _row_spec(),
                  _rep_spec((H, H)), _rep_spec((1, H)), _rep_spec((1, H)),
                  _rep_spec((1, H)),
                  _rep_spec((H, D)), _rep_spec((1, D))],
        out_specs=_row_spec(),
        out_shape=jax.ShapeDtypeStruct((N, D), jnp.float32),
    )(x, a0, a1, W2, b2.reshape(1, H), g2.reshape(1, H), be2.reshape(1, H),
      Wf, bf.reshape(1, D))


def kernel(node_features, adj_indices, adj_values, W1, b1, g1, be1,
           W2, b2, g2, be2, Wf, bf):
    dst3d = adj_indices[0].astype(jnp.int32).reshape(NCHUNK, 1, C)
    src3d = adj_indices[1].astype(jnp.int32).reshape(NCHUNK, 1, C)
    val3d = adj_values.reshape(NCHUNK, 1, C)

    a0, a1 = _sc_aggregate(node_features, dst3d, src3d, val3d)
    x1 = _dense1(node_features, a0, a1, W1, b1, g1, be1)
    c0, c1 = _sc_aggregate(x1, dst3d, src3d, val3d)
    return _dense2(x1, c0, c1, W2, b2, g2, be2, Wf, bf)
